# baseline (speedup 1.0000x reference)
import functools

import jax
import jax.numpy as jnp
from jax import lax
from jax.experimental import pallas as pl
from jax.experimental.pallas import tpu as pltpu
from jax.experimental.pallas import tpu_sc as plsc

N = 10000
E = 320000
D_IN = 128
D_HID = 16
D_OUT = 64
MU = 0.1
P = 2.0

NS = 16
CH = 512
NCH = E // CH
CH_W = 40
NCH_P = NS * CH_W
SL = 640
NP = NS * SL
G = 2
M = CH_W // G

def _deg_body(col2d, deg_out, deg_sh, cidx, ones_v, zv, sem):
    w = lax.axis_index("s")
    n0 = w * SL
    e0 = w * CH_W

    pltpu.sync_copy(col2d.at[pl.ds(e0, CH_W)], cidx)

    def _fill(i, _):
        ones_v[pl.ds(i * 16, 16)] = jnp.ones((16,), jnp.float32)
        zv[pl.ds(i * 16, 16)] = jnp.zeros((16,), jnp.float32)
        return 0
    lax.fori_loop(0, CH // 16, _fill, 0)

    def _zero(i, _):
        zv[pl.ds(CH + i * 16, 16)] = jnp.zeros((16,), jnp.float32)
        return 0
    lax.fori_loop(0, (SL - CH) // 16, _zero, 0)

    pltpu.sync_copy(zv, deg_sh.at[pl.ds(n0, SL)])
    plsc.subcore_barrier()

    def _chunk(j, _):
        pltpu.sync_copy(ones_v, deg_sh.at[cidx.at[j]], add=True)
        return 0
    lax.fori_loop(0, CH_W, _chunk, 0)
    plsc.subcore_barrier()

    pltpu.sync_copy(deg_sh.at[pl.ds(n0, SL)], deg_out.at[pl.ds(n0, SL)])


@functools.lru_cache(maxsize=None)
def _get_sc_deg():
    mesh = plsc.VectorSubcoreMesh(
        core_axis_name="c", subcore_axis_name="s",
        num_cores=1, num_subcores=NS)
    return pl.kernel(
        _deg_body,
        out_type=jax.ShapeDtypeStruct((NP,), jnp.float32),
        mesh=mesh,
        compiler_params=pltpu.CompilerParams(use_tc_tiling_on_sc=False),
        scratch_types=[
            pltpu.VMEM_SHARED((NP,), jnp.float32),
            pltpu.VMEM((CH_W, CH), jnp.int32),
            pltpu.VMEM((CH,), jnp.float32),
            pltpu.VMEM((SL,), jnp.float32),
            pltpu.SemaphoreType.DMA,
        ],
    )


def _edge_pass(src, t_sh, ridx, cidx, rows, sgA, sgB, ssA, ssB):
    def issue_g(m, base, sem):
        for i in range(G):
            pltpu.async_copy(src.at[ridx.at[m * G + i]], rows.at[base + i],
                             sem)

    def drain_g(m, base, sem):
        for i in range(G):
            pltpu.make_async_copy(src.at[ridx.at[m * G + i]],
                                  rows.at[base + i], sem).wait()

    def issue_s(m, base, sem):
        for i in range(G):
            pltpu.async_copy(rows.at[base + i], t_sh.at[cidx.at[m * G + i]],
                             sem, add=True)

    def drain_s(m, base, sem):
        for i in range(G):
            pltpu.make_async_copy(rows.at[base + i],
                                  t_sh.at[cidx.at[m * G + i]], sem).wait()

    issue_g(0, 0, sgA)

    def body(k, _):
        mA = 2 * k
        mB = 2 * k + 1
        issue_g(mB, G, sgB)
        drain_g(mA, 0, sgA)
        issue_s(mA, 0, ssA)
        drain_g(mB, G, sgB)
        issue_s(mB, G, ssB)
        drain_s(mA, 0, ssA)

        @pl.when(k < M // 2 - 1)
        def _():
            issue_g(mA + 2, 0, sgA)

        drain_s(mB, G, ssB)
        return 0
    lax.fori_loop(0, M // 2, body, 0)


def _loop_body(s1, c1, u, row2d, col2d, t2_out, s2_hbm,
               t_sh, ridx, cidx, rows, tv, uv, sv, c1v,
               sgA, sgB, ssA, ssB):
    w = lax.axis_index("s")
    n0 = w * SL
    e0 = w * CH_W

    pltpu.sync_copy(row2d.at[pl.ds(e0, CH_W)], ridx)
    pltpu.sync_copy(col2d.at[pl.ds(e0, CH_W)], cidx)

    def _zrow(i, _):
        sv[i] = jnp.zeros((16,), jnp.float32)
        return 0
    lax.fori_loop(0, SL, _zrow, 0)
    pltpu.sync_copy(sv, t_sh.at[pl.ds(n0, SL)])
    plsc.subcore_barrier()

    _edge_pass(s1, t_sh, ridx, cidx, rows, sgA, sgB, ssA, ssB)
    plsc.subcore_barrier()

    pltpu.sync_copy(t_sh.at[pl.ds(n0, SL)], tv)
    pltpu.sync_copy(u.at[pl.ds(n0, SL)], uv)
    pltpu.sync_copy(c1.at[pl.ds(n0, SL)], c1v)

    def _nblock(b, _):
        r0 = b * 16
        for jj in range(16):
            sv[r0 + jj] = c1v[r0 + jj] * tv[r0 + jj] + uv[r0 + jj]
        return 0
    lax.fori_loop(0, SL // 16, _nblock, 0)
    pltpu.sync_copy(sv, s2_hbm.at[pl.ds(n0, SL)])

    def _zrow2(i, _):
        tv[i] = jnp.zeros((16,), jnp.float32)
        return 0
    lax.fori_loop(0, SL, _zrow2, 0)
    pltpu.sync_copy(tv, t_sh.at[pl.ds(n0, SL)])
    plsc.subcore_barrier()

    _edge_pass(s2_hbm, t_sh, ridx, cidx, rows, sgA, sgB, ssA, ssB)
    plsc.subcore_barrier()

    pltpu.sync_copy(t_sh.at[pl.ds(n0, SL)], t2_out.at[pl.ds(n0, SL)])


@functools.lru_cache(maxsize=None)
def _get_sc_loop():
    mesh = plsc.VectorSubcoreMesh(
        core_axis_name="c", subcore_axis_name="s",
        num_cores=1, num_subcores=NS)
    return pl.kernel(
        _loop_body,
        out_type=(jax.ShapeDtypeStruct((NP, D_HID), jnp.float32),
                  jax.ShapeDtypeStruct((NP, D_HID), jnp.float32)),
        mesh=mesh,
        compiler_params=pltpu.CompilerParams(use_tc_tiling_on_sc=False),
        scratch_types=[
        pltpu.VMEM_SHARED((NP, D_HID), jnp.float32),
        pltpu.VMEM((CH_W, CH), jnp.int32),
        pltpu.VMEM((CH_W, CH), jnp.int32),
        pltpu.VMEM((2 * G, CH, D_HID), jnp.float32),
        pltpu.VMEM((SL, D_HID), jnp.float32),
        pltpu.VMEM((SL, D_HID), jnp.float32),
        pltpu.VMEM((SL, D_HID), jnp.float32),
        pltpu.VMEM((SL, D_HID), jnp.float32),
        pltpu.SemaphoreType.DMA,
        pltpu.SemaphoreType.DMA,
        pltpu.SemaphoreType.DMA,
        pltpu.SemaphoreType.DMA,
    ],
    )


def _pre_body(x_ref, w1_ref, b1_ref, deg_ref,
              h_ref, s1_ref, u_ref, c1_ref):
    h = jnp.maximum(
        jnp.dot(x_ref[...], w1_ref[...],
                preferred_element_type=jnp.float32) + b1_ref[...], 0.0)
    deg = deg_ref[...]
    dis = jnp.where(deg > 0, lax.rsqrt(deg), 0.0)
    alpha = 1.0 / (dis * dis * deg + 2.0 * MU / P)
    beta = (4.0 * MU / P) * alpha
    a = alpha * dis
    h_ref[...] = h
    s1_ref[...] = a * h
    u_ref[...] = (a * beta) * h
    c1_ref[...] = jnp.broadcast_to(alpha * dis * dis, h.shape)


_tc_pre = pl.pallas_call(
    _pre_body,
    out_shape=(jax.ShapeDtypeStruct((N, D_HID), jnp.float32),
               jax.ShapeDtypeStruct((N, D_HID), jnp.float32),
               jax.ShapeDtypeStruct((N, D_HID), jnp.float32),
               jax.ShapeDtypeStruct((N, D_HID), jnp.float32)),
)


def _post_body(t2_ref, deg_ref, h_ref, w2_ref, b2_ref, out_ref):
    deg = deg_ref[...]
    dis = jnp.where(deg > 0, lax.rsqrt(deg), 0.0)
    alpha = 1.0 / (dis * dis * deg + 2.0 * MU / P)
    beta = (4.0 * MU / P) * alpha
    out2 = dis * t2_ref[...] + beta * h_ref[...]
    logits = jnp.dot(out2, w2_ref[...],
                     preferred_element_type=jnp.float32) + b2_ref[...]
    m = jnp.max(logits, axis=1, keepdims=True)
    lse = jnp.log(jnp.sum(jnp.exp(logits - m), axis=1, keepdims=True)) + m
    out_ref[...] = logits - lse


_tc_post = pl.pallas_call(
    _post_body,
    out_shape=jax.ShapeDtypeStruct((N, D_OUT), jnp.float32),
)


def kernel(x, W1, b1, W2, b2, edge_index):
    epad = ((0, NCH_P - NCH), (0, 0))
    row2d = jnp.pad(edge_index[0].reshape(NCH, CH), epad,
                    constant_values=NP - 1)
    col2d = jnp.pad(edge_index[1].reshape(NCH, CH), epad,
                    constant_values=NP - 1)

    deg_p = _get_sc_deg()(col2d)
    deg = deg_p[:N].reshape(N, 1)

    h, s1, u, c1 = _tc_pre(x, W1, b1.reshape(1, D_HID), deg)

    pad = ((0, NP - N), (0, 0))
    t2p, _ = _get_sc_loop()(jnp.pad(s1, pad), jnp.pad(c1, pad),
                            jnp.pad(u, pad), row2d, col2d)

    return _tc_post(t2p[:N], deg, h, W2, b2.reshape(1, D_OUT))


# speedup vs baseline: 1.3924x; 1.3924x over previous
import functools

import jax
import jax.numpy as jnp
from jax import lax
from jax.experimental import pallas as pl
from jax.experimental.pallas import tpu as pltpu
from jax.experimental.pallas import tpu_sc as plsc

N = 10000
E = 320000
D_IN = 128
D_HID = 16
D_OUT = 64
MU = 0.1
P = 2.0

NS = 16
CH = 512
NCH = E // CH
CH_W = 40
NCH_P = NS * CH_W
SL = 640
NP = NS * SL
G = 2
M = CH_W // G

def _deg_body(col2d, deg_out, deg_sh, cidx, ones_v, zv, sem):
    w = lax.axis_index("s")
    n0 = w * SL
    e0 = w * CH_W

    pltpu.sync_copy(col2d.at[pl.ds(e0, CH_W)], cidx)

    def _fill(i, _):
        ones_v[pl.ds(i * 16, 16)] = jnp.ones((16,), jnp.float32)
        zv[pl.ds(i * 16, 16)] = jnp.zeros((16,), jnp.float32)
        return 0
    lax.fori_loop(0, CH // 16, _fill, 0)

    def _zero(i, _):
        zv[pl.ds(CH + i * 16, 16)] = jnp.zeros((16,), jnp.float32)
        return 0
    lax.fori_loop(0, (SL - CH) // 16, _zero, 0)

    pltpu.sync_copy(zv, deg_sh.at[pl.ds(n0, SL)])
    plsc.subcore_barrier()

    def _chunk(j, _):
        pltpu.sync_copy(ones_v, deg_sh.at[cidx.at[j]], add=True)
        return 0
    lax.fori_loop(0, CH_W, _chunk, 0)
    plsc.subcore_barrier()

    pltpu.sync_copy(deg_sh.at[pl.ds(n0, SL)], deg_out.at[pl.ds(n0, SL)])


@functools.lru_cache(maxsize=None)
def _get_sc_deg():
    mesh = plsc.VectorSubcoreMesh(
        core_axis_name="c", subcore_axis_name="s",
        num_cores=1, num_subcores=NS)
    return pl.kernel(
        _deg_body,
        out_type=jax.ShapeDtypeStruct((NP,), jnp.float32),
        mesh=mesh,
        compiler_params=pltpu.CompilerParams(use_tc_tiling_on_sc=False),
        scratch_types=[
            pltpu.VMEM_SHARED((NP,), jnp.float32),
            pltpu.VMEM((CH_W, CH), jnp.int32),
            pltpu.VMEM((CH,), jnp.float32),
            pltpu.VMEM((SL,), jnp.float32),
            pltpu.SemaphoreType.DMA,
        ],
    )


def _edge_pass(src, t_sh, ridx, cidx, rows, sgA, sgB, ssA, ssB):
    def issue_g(m, base, sem):
        for i in range(G):
            pltpu.async_copy(src.at[ridx.at[m * G + i]], rows.at[base + i],
                             sem)

    def drain_g(m, base, sem):
        for i in range(G):
            pltpu.make_async_copy(src.at[ridx.at[m * G + i]],
                                  rows.at[base + i], sem).wait()

    def issue_s(m, base, sem):
        for i in range(G):
            pltpu.async_copy(rows.at[base + i], t_sh.at[cidx.at[m * G + i]],
                             sem, add=True)

    def drain_s(m, base, sem):
        for i in range(G):
            pltpu.make_async_copy(rows.at[base + i],
                                  t_sh.at[cidx.at[m * G + i]], sem).wait()

    issue_g(0, 0, sgA)

    def body(k, _):
        mA = 2 * k
        mB = 2 * k + 1
        issue_g(mB, G, sgB)
        drain_g(mA, 0, sgA)
        issue_s(mA, 0, ssA)
        drain_s(mA, 0, ssA)

        @pl.when(k < M // 2 - 1)
        def _():
            issue_g(mA + 2, 0, sgA)

        drain_g(mB, G, sgB)
        issue_s(mB, G, ssB)
        drain_s(mB, G, ssB)
        return 0
    lax.fori_loop(0, M // 2, body, 0)


def _loop_body(s1, c1, uoc, row2d, col2d, t2_out,
               t_sh, s_sh, ridx, cidx, rows, tv, sv, c1v,
               sgA, sgB, ssA, ssB):
    w = lax.axis_index("s")
    n0 = w * SL
    e0 = w * CH_W

    pltpu.sync_copy(row2d.at[pl.ds(e0, CH_W)], ridx)
    pltpu.sync_copy(col2d.at[pl.ds(e0, CH_W)], cidx)

    pltpu.sync_copy(s1.at[pl.ds(n0, SL)], s_sh.at[pl.ds(n0, SL)])
    pltpu.sync_copy(uoc.at[pl.ds(n0, SL)], t_sh.at[pl.ds(n0, SL)])
    pltpu.sync_copy(c1.at[pl.ds(n0, SL)], c1v)
    plsc.subcore_barrier()

    _edge_pass(s_sh, t_sh, ridx, cidx, rows, sgA, sgB, ssA, ssB)
    plsc.subcore_barrier()

    pltpu.sync_copy(t_sh.at[pl.ds(n0, SL)], tv)

    def _nblock(b, _):
        r0 = b * 16
        for jj in range(16):
            sv[r0 + jj] = c1v[r0 + jj] * tv[r0 + jj]
        return 0
    lax.fori_loop(0, SL // 16, _nblock, 0)
    pltpu.sync_copy(sv, s_sh.at[pl.ds(n0, SL)])

    def _zrow2(i, _):
        tv[i] = jnp.zeros((16,), jnp.float32)
        return 0
    lax.fori_loop(0, SL, _zrow2, 0)
    pltpu.sync_copy(tv, t_sh.at[pl.ds(n0, SL)])
    plsc.subcore_barrier()

    _edge_pass(s_sh, t_sh, ridx, cidx, rows, sgA, sgB, ssA, ssB)
    plsc.subcore_barrier()

    pltpu.sync_copy(t_sh.at[pl.ds(n0, SL)], t2_out.at[pl.ds(n0, SL)])


@functools.lru_cache(maxsize=None)
def _get_sc_loop():
    mesh = plsc.VectorSubcoreMesh(
        core_axis_name="c", subcore_axis_name="s",
        num_cores=1, num_subcores=NS)
    return pl.kernel(
        _loop_body,
        out_type=jax.ShapeDtypeStruct((NP, D_HID), jnp.float32),
        mesh=mesh,
        compiler_params=pltpu.CompilerParams(use_tc_tiling_on_sc=False),
        scratch_types=[
        pltpu.VMEM_SHARED((NP, D_HID), jnp.float32),
        pltpu.VMEM_SHARED((NP, D_HID), jnp.float32),
        pltpu.VMEM((CH_W, CH), jnp.int32),
        pltpu.VMEM((CH_W, CH), jnp.int32),
        pltpu.VMEM((2 * G, CH, D_HID), jnp.float32),
        pltpu.VMEM((SL, D_HID), jnp.float32),
        pltpu.VMEM((SL, D_HID), jnp.float32),
        pltpu.VMEM((SL, D_HID), jnp.float32),
        pltpu.SemaphoreType.DMA,
        pltpu.SemaphoreType.DMA,
        pltpu.SemaphoreType.DMA,
        pltpu.SemaphoreType.DMA,
    ],
    )


def _pre_body(x_ref, w1_ref, b1_ref, deg_ref,
              h_ref, s1_ref, u_ref, c1_ref):
    h = jnp.maximum(
        jnp.dot(x_ref[...], w1_ref[...],
                preferred_element_type=jnp.float32) + b1_ref[...], 0.0)
    deg = deg_ref[...]
    dis = jnp.where(deg > 0, lax.rsqrt(deg), 0.0)
    alpha = 1.0 / (dis * dis * deg + 2.0 * MU / P)
    beta = (4.0 * MU / P) * alpha
    a = alpha * dis
    c1 = alpha * dis * dis
    u = (a * beta) * h
    h_ref[...] = h
    s1_ref[...] = a * h
    u_ref[...] = jnp.where(c1 > 0, u / c1, 0.0)
    c1_ref[...] = jnp.broadcast_to(c1, h.shape)


_tc_pre = pl.pallas_call(
    _pre_body,
    out_shape=(jax.ShapeDtypeStruct((N, D_HID), jnp.float32),
               jax.ShapeDtypeStruct((N, D_HID), jnp.float32),
               jax.ShapeDtypeStruct((N, D_HID), jnp.float32),
               jax.ShapeDtypeStruct((N, D_HID), jnp.float32)),
)


def _post_body(t2_ref, deg_ref, h_ref, w2_ref, b2_ref, out_ref):
    deg = deg_ref[...]
    dis = jnp.where(deg > 0, lax.rsqrt(deg), 0.0)
    alpha = 1.0 / (dis * dis * deg + 2.0 * MU / P)
    beta = (4.0 * MU / P) * alpha
    out2 = dis * t2_ref[...] + beta * h_ref[...]
    logits = jnp.dot(out2, w2_ref[...],
                     preferred_element_type=jnp.float32) + b2_ref[...]
    m = jnp.max(logits, axis=1, keepdims=True)
    lse = jnp.log(jnp.sum(jnp.exp(logits - m), axis=1, keepdims=True)) + m
    out_ref[...] = logits - lse


_tc_post = pl.pallas_call(
    _post_body,
    out_shape=jax.ShapeDtypeStruct((N, D_OUT), jnp.float32),
)


def kernel(x, W1, b1, W2, b2, edge_index):
    epad = ((0, NCH_P - NCH), (0, 0))
    row2d = jnp.pad(edge_index[0].reshape(NCH, CH), epad,
                    constant_values=NP - 1)
    col2d = jnp.pad(edge_index[1].reshape(NCH, CH), epad,
                    constant_values=NP - 1)

    deg_p = _get_sc_deg()(col2d)
    deg = deg_p[:N].reshape(N, 1)

    h, s1, uoc, c1 = _tc_pre(x, W1, b1.reshape(1, D_HID), deg)

    pad = ((0, NP - N), (0, 0))
    t2p = _get_sc_loop()(jnp.pad(s1, pad), jnp.pad(c1, pad),
                         jnp.pad(uoc, pad), row2d, col2d)

    return _tc_post(t2p[:N], deg, h, W2, b2.reshape(1, D_OUT))


# speedup vs baseline: 1.4342x; 1.0300x over previous
import functools

import jax
import jax.numpy as jnp
from jax import lax
from jax.experimental import pallas as pl
from jax.experimental.pallas import tpu as pltpu
from jax.experimental.pallas import tpu_sc as plsc

N = 10000
E = 320000
D_IN = 128
D_HID = 16
D_OUT = 64
MU = 0.1
P = 2.0

NS = 16
CH = 512
NCH = E // CH
CH_W = 40
NCH_P = NS * CH_W
SL = 640
NP = NS * SL
G = 2
M = CH_W // G

C0 = 2.0 * MU / P
C1 = 4.0 * MU / P


def _rsqrt16(x):
    i = lax.bitcast_convert_type(x, jnp.int32)
    i = jnp.int32(0x5F3759DF) - (i >> 1)
    y = lax.bitcast_convert_type(i, jnp.float32)
    for _ in range(4):
        y = y * (1.5 - 0.5 * x * y * y)
    return y


def _coeffs(d16):
    pos = d16 > 0.0
    dis = jnp.where(pos, _rsqrt16(d16), 0.0)
    alpha = 1.0 / (dis * dis * d16 + C0)
    beta = C1 * alpha
    return pos, dis, alpha, beta


def _edge_pass(src, t_sh, ridx, cidx, rows, sgA, sgB, ssA, ssB):
    def issue_g(m, base, sem):
        for i in range(G):
            pltpu.async_copy(src.at[ridx.at[m * G + i]], rows.at[base + i],
                             sem)

    def drain_g(m, base, sem):
        for i in range(G):
            pltpu.make_async_copy(src.at[ridx.at[m * G + i]],
                                  rows.at[base + i], sem).wait()

    def issue_s(m, base, sem):
        for i in range(G):
            pltpu.async_copy(rows.at[base + i], t_sh.at[cidx.at[m * G + i]],
                             sem, add=True)

    def drain_s(m, base, sem):
        for i in range(G):
            pltpu.make_async_copy(rows.at[base + i],
                                  t_sh.at[cidx.at[m * G + i]], sem).wait()

    issue_g(0, 0, sgA)

    def body(k, _):
        mA = 2 * k
        mB = 2 * k + 1
        issue_g(mB, G, sgB)
        drain_g(mA, 0, sgA)
        issue_s(mA, 0, ssA)
        drain_s(mA, 0, ssA)

        @pl.when(k < M // 2 - 1)
        def _():
            issue_g(mA + 2, 0, sgA)

        drain_g(mB, G, sgB)
        issue_s(mB, G, ssB)
        drain_s(mB, G, ssB)
        return 0
    lax.fori_loop(0, M // 2, body, 0)


def _main_body(h_p, row2d, col2d, out2,
               t_sh, s_sh, ridx, cidx, rows, tv, degv, hv,
               sgA, sgB, ssA, ssB):
    w = lax.axis_index("s")
    n0 = w * SL
    e0 = w * CH_W

    pltpu.sync_copy(row2d.at[pl.ds(e0, CH_W)], ridx)
    pltpu.sync_copy(col2d.at[pl.ds(e0, CH_W)], cidx)
    pltpu.sync_copy(h_p.at[pl.ds(n0, SL)], hv)

    ones_rows = rows.at[0]

    def _fill(i, _):
        ones_rows[i] = jnp.ones((16,), jnp.float32)
        tv[i] = jnp.zeros((16,), jnp.float32)
        return 0
    lax.fori_loop(0, CH, _fill, 0)

    def _fill2(i, _):
        tv[CH + i] = jnp.zeros((16,), jnp.float32)
        return 0
    lax.fori_loop(0, SL - CH, _fill2, 0)
    pltpu.sync_copy(tv, t_sh.at[pl.ds(n0, SL)])
    plsc.subcore_barrier()

    def _dchunk(k, _):
        for b in range(4):
            pltpu.async_copy(ones_rows, t_sh.at[cidx.at[4 * k + b]],
                             ssA, add=True)
        for b in range(4):
            pltpu.make_async_copy(ones_rows, t_sh.at[cidx.at[4 * k + b]],
                                  ssA).wait()
        return 0
    lax.fori_loop(0, CH_W // 4, _dchunk, 0)
    plsc.subcore_barrier()

    pltpu.sync_copy(t_sh.at[pl.ds(n0, SL)], degv)

    def _s1(i, _):
        _, dis, alpha, _ = _coeffs(degv[i])
        tv[i] = (alpha * dis) * hv[i]
        return 0
    lax.fori_loop(0, SL, _s1, 0)
    pltpu.sync_copy(tv, s_sh.at[pl.ds(n0, SL)])

    def _uoc(i, _):
        _, dis, _, beta = _coeffs(degv[i])
        d16 = degv[i]
        tv[i] = beta * d16 * dis * hv[i]
        return 0
    lax.fori_loop(0, SL, _uoc, 0)
    pltpu.sync_copy(tv, t_sh.at[pl.ds(n0, SL)])
    plsc.subcore_barrier()

    _edge_pass(s_sh, t_sh, ridx, cidx, rows, sgA, sgB, ssA, ssB)
    plsc.subcore_barrier()

    pltpu.sync_copy(t_sh.at[pl.ds(n0, SL)], tv)

    def _s2(i, _):
        _, dis, alpha, _ = _coeffs(degv[i])
        tv[i] = (alpha * dis * dis) * tv[i]
        return 0
    lax.fori_loop(0, SL, _s2, 0)
    pltpu.sync_copy(tv, s_sh.at[pl.ds(n0, SL)])

    def _uoc2(i, _):
        _, dis, _, beta = _coeffs(degv[i])
        d16 = degv[i]
        tv[i] = beta * d16 * dis * hv[i]
        return 0
    lax.fori_loop(0, SL, _uoc2, 0)
    pltpu.sync_copy(tv, t_sh.at[pl.ds(n0, SL)])
    plsc.subcore_barrier()

    _edge_pass(s_sh, t_sh, ridx, cidx, rows, sgA, sgB, ssA, ssB)
    plsc.subcore_barrier()

    pltpu.sync_copy(t_sh.at[pl.ds(n0, SL)], tv)

    def _fin(i, _):
        pos, dis, _, beta = _coeffs(degv[i])
        tv[i] = jnp.where(pos, dis * tv[i], beta * hv[i])
        return 0
    lax.fori_loop(0, SL, _fin, 0)
    pltpu.sync_copy(tv, out2.at[pl.ds(n0, SL)])


@functools.lru_cache(maxsize=None)
def _get_sc_main():
    mesh = plsc.VectorSubcoreMesh(
        core_axis_name="c", subcore_axis_name="s",
        num_cores=1, num_subcores=NS)
    return pl.kernel(
        _main_body,
        out_type=jax.ShapeDtypeStruct((NP, D_HID), jnp.float32),
        mesh=mesh,
        compiler_params=pltpu.CompilerParams(use_tc_tiling_on_sc=False),
        scratch_types=[
            pltpu.VMEM_SHARED((NP, D_HID), jnp.float32),
            pltpu.VMEM_SHARED((NP, D_HID), jnp.float32),
            pltpu.VMEM((CH_W, CH), jnp.int32),
            pltpu.VMEM((CH_W, CH), jnp.int32),
            pltpu.VMEM((2 * G, CH, D_HID), jnp.float32),
            pltpu.VMEM((SL, D_HID), jnp.float32),
            pltpu.VMEM((SL, D_HID), jnp.float32),
            pltpu.VMEM((SL, D_HID), jnp.float32),
            pltpu.SemaphoreType.DMA,
            pltpu.SemaphoreType.DMA,
            pltpu.SemaphoreType.DMA,
            pltpu.SemaphoreType.DMA,
        ],
    )


def _pre_body(x_ref, w1_ref, b1_ref, h_ref):
    h_ref[...] = jnp.maximum(
        jnp.dot(x_ref[...], w1_ref[...],
                preferred_element_type=jnp.float32) + b1_ref[...], 0.0)


_tc_pre = pl.pallas_call(
    _pre_body,
    out_shape=jax.ShapeDtypeStruct((N, D_HID), jnp.float32),
)


def _post_body(o_ref, w2_ref, b2_ref, out_ref):
    logits = jnp.dot(o_ref[...], w2_ref[...],
                     preferred_element_type=jnp.float32) + b2_ref[...]
    m = jnp.max(logits, axis=1, keepdims=True)
    lse = jnp.log(jnp.sum(jnp.exp(logits - m), axis=1, keepdims=True)) + m
    out_ref[...] = logits - lse


_tc_post = pl.pallas_call(
    _post_body,
    out_shape=jax.ShapeDtypeStruct((N, D_OUT), jnp.float32),
)


def kernel(x, W1, b1, W2, b2, edge_index):
    epad = ((0, NCH_P - NCH), (0, 0))
    row2d = jnp.pad(edge_index[0].reshape(NCH, CH), epad,
                    constant_values=NP - 1)
    col2d = jnp.pad(edge_index[1].reshape(NCH, CH), epad,
                    constant_values=NP - 1)

    h = _tc_pre(x, W1, b1.reshape(1, D_HID))
    h_p = jnp.pad(h, ((0, NP - N), (0, 0)))

    out2 = _get_sc_main()(h_p, row2d, col2d)

    return _tc_post(out2[:N], W2, b2.reshape(1, D_OUT))
